# baseline (speedup 1.0000x reference)
import functools

import jax
import jax.numpy as jnp
from jax import lax
from jax.experimental import pallas as pl
from jax.experimental.pallas import tpu as pltpu
from jax.experimental.pallas import tpu_sc as plsc

B = 16384
L = 20
D = 16
LANES = 16


def _allsum(v):
    lane = lax.iota(jnp.int32, LANES)
    dn = lax.GatherDimensionNumbers(
        offset_dims=(), collapsed_slice_dims=(0,), start_index_map=(0,))
    for sh in (8, 4, 2, 1):
        perm = lax.gather(v, (lane ^ sh)[:, None], dn, slice_sizes=(1,),
                          mode=lax.GatherScatterMode.PROMISE_IN_BOUNDS)
        v = v + perm
    return v


def _rsqrt_vec(x):
    i = lax.bitcast_convert_type(x, jnp.int32)
    i = jnp.int32(0x5F3759DF) - lax.shift_right_logical(i, 1)
    y = lax.bitcast_convert_type(i, jnp.float32)
    y = y * (1.5 - 0.5 * x * y * y)
    y = y * (1.5 - 0.5 * x * y * y)
    return y


def _make_encoder(nc, ns):
    nw = nc * ns
    pw = B // nw
    ch = 128
    chunks = pw // ch
    lp = 24
    rows_per_chunk = ch * lp

    mesh = plsc.VectorSubcoreMesh(core_axis_name="c", subcore_axis_name="s")

    @functools.partial(
        pl.kernel,
        out_type=jax.ShapeDtypeStruct((B, D), jnp.float32),
        mesh=mesh,
        compiler_params=pltpu.CompilerParams(use_tc_tiling_on_sc=False),
        scratch_types=[
            pltpu.VMEM((pw, lp), jnp.int32),
            pltpu.VMEM((2, rows_per_chunk, D), jnp.float32),
            pltpu.VMEM((ch, D), jnp.float32),
            pltpu.SemaphoreType.DMA((2,)),
        ],
    )
    def encode(idx_hbm, table_hbm, out_hbm, idx_v, rows_v, out_v, sem):
        wid = lax.axis_index("s") * nc + lax.axis_index("c")
        base_pt0 = pl.multiple_of(wid * pw, 8)
        pltpu.sync_copy(idx_hbm.at[pl.ds(base_pt0, pw)], idx_v)

        def issue(c):
            buf = c % 2

            def gather_one(p, carry):
                dst = rows_v.at[buf, pl.ds(pl.multiple_of(p * lp, 8), lp)]
                pltpu.async_copy(
                    table_hbm.at[idx_v.at[c * ch + p]], dst, sem.at[buf])
                return carry

            lax.fori_loop(0, ch, gather_one, 0)

        def drain(c):
            buf = c % 2
            pltpu.make_async_copy(
                table_hbm.at[pl.ds(0, rows_per_chunk)], rows_v.at[buf],
                sem.at[buf]).wait()

        issue(0)
        for c in range(chunks):
            if c + 1 < chunks:
                issue(c + 1)
            drain(c)
            buf = c % 2

            def point_body(p, carry, buf=buf):
                rbase = p * lp
                acc = jnp.zeros((LANES,), jnp.float32)
                for j in range(L):
                    v = rows_v[buf, rbase + j]
                    acc = acc + v * _rsqrt_vec(_allsum(v * v))
                s2 = _allsum(acc * acc)
                out_v[p] = acc * _rsqrt_vec(s2)
                return carry

            lax.fori_loop(0, ch, point_body, 0)
            base_pt = pl.multiple_of(wid * pw + c * ch, 8)
            pltpu.sync_copy(out_v, out_hbm.at[pl.ds(base_pt, ch)])

    return encode


def kernel(indices, table):
    info = plsc.get_sparse_core_info()
    enc = _make_encoder(info.num_cores, info.num_subcores)
    idx24 = jnp.pad(indices.astype(jnp.int32), ((0, 0), (0, 4)))
    return enc(idx24, table)


# speedup vs baseline: 4.0173x; 4.0173x over previous
import functools

import jax
import jax.numpy as jnp
from jax import lax
from jax.experimental import pallas as pl
from jax.experimental.pallas import tpu as pltpu
from jax.experimental.pallas import tpu_sc as plsc

B = 16384
L = 20
D = 16
LANES = 16
VOCAB = 1000000
VMAIN = 999936
NBLK = VMAIN // 256
BPW = 124


def _lane():
    return lax.iota(jnp.int32, LANES)


def _perm(v, idx):
    dn = lax.GatherDimensionNumbers(
        offset_dims=(), collapsed_slice_dims=(0,), start_index_map=(0,))
    return lax.gather(v, idx[:, None], dn, slice_sizes=(1,),
                      mode=lax.GatherScatterMode.PROMISE_IN_BOUNDS)


def _allreduce(v, op):
    lane = _lane()
    for sh in (8, 4, 2, 1):
        v = op(v, _perm(v, lane ^ sh))
    return v


def _rsqrt_vec(x, iters=2):
    i = lax.bitcast_convert_type(x, jnp.int32)
    i = jnp.int32(0x5F3759DF) - lax.shift_right_logical(i, 1)
    y = lax.bitcast_convert_type(i, jnp.float32)
    for _ in range(iters):
        y = y * (1.5 - 0.5 * x * y * y)
    return y


def _make_transpose(nc, ns):
    mesh = plsc.VectorSubcoreMesh(core_axis_name="c", subcore_axis_name="s")

    @functools.partial(
        pl.kernel,
        out_type=jax.ShapeDtypeStruct((VOCAB // 8, 128), jnp.float32),
        mesh=mesh,
        compiler_params=pltpu.CompilerParams(use_tc_tiling_on_sc=True),
        scratch_types=[
            pltpu.VMEM((2, 16, 256), jnp.float32),
            pltpu.VMEM((2, 32, 128), jnp.float32),
            pltpu.VMEM((8, 128), jnp.float32),
            pltpu.SemaphoreType.DMA((2,)),
            pltpu.SemaphoreType.DMA((2,)),
        ],
    )
    def transpose_k(tt_hbm, taux_hbm, out_hbm, in2, ou2, tb_v, sem_i, sem_o):
        wid = lax.axis_index("s") * nc + lax.axis_index("c")

        @pl.when(wid == nc * ns - 1)
        def _():
            pltpu.sync_copy(taux_hbm, tb_v)
            pltpu.sync_copy(tb_v, out_hbm.at[pl.ds(VMAIN // 8, 8)])

        start = jnp.minimum(wid * 122 + jnp.minimum(wid, 2), NBLK - BPW)

        lane = _lane()
        perms = {sh: lane ^ sh for sh in (8, 4, 2, 1)}
        masks = {sh: (lane & sh) == 0 for sh in (8, 4, 2, 1)}

        def issue_in(q, b):
            off = pl.multiple_of(b * 256, 128)
            pltpu.async_copy(tt_hbm.at[:, pl.ds(off, 256)], in2.at[q],
                             sem_i.at[q])

        def drain_in(q):
            pltpu.make_async_copy(tt_hbm.at[:, pl.ds(0, 256)], in2.at[q],
                                  sem_i.at[q]).wait()

        def issue_out(q, b):
            off = pl.multiple_of(b * 32, 8)
            pltpu.async_copy(ou2.at[q], out_hbm.at[pl.ds(off, 32)],
                             sem_o.at[q])

        def drain_out(q):
            pltpu.make_async_copy(ou2.at[q], out_hbm.at[pl.ds(0, 32)],
                                  sem_o.at[q]).wait()

        def do_transpose(q):
            for s in range(16):
                r = [in2[q, d, pl.ds(16 * s, 16)] for d in range(16)]
                for sh in (8, 4, 2, 1):
                    pi, m = perms[sh], masks[sh]
                    for d in range(16):
                        if d & sh:
                            continue
                        a, b = r[d], r[d | sh]
                        pa, pb = _perm(a, pi), _perm(b, pi)
                        r[d] = jnp.where(m, a, pb)
                        r[d | sh] = jnp.where(m, pa, b)
                for vq in range(16):
                    v = 16 * s + vq
                    ou2[q, v // 8, pl.ds((v % 8) * 16, 16)] = r[vq]

        issue_in(0, start)
        issue_in(1, start + 1)

        def body(it, carry):
            for q in (0, 1):
                b = start + 2 * it + q
                drain_in(q)

                @pl.when(it > 0)
                def _():
                    drain_out(q)

                do_transpose(q)
                issue_out(q, b)
                issue_in(q, jnp.minimum(b + 2, NBLK - 1))
            return carry

        lax.fori_loop(0, BPW // 2, body, 0)
        drain_in(0)
        drain_in(1)
        drain_out(0)
        drain_out(1)

    return transpose_k


def _make_encoder(nc, ns):
    nw = nc * ns
    pw = B // nw
    ch = 256
    chunks = pw // ch
    rows_per_chunk = ch * L
    g_per_chunk = rows_per_chunk // 128

    mesh = plsc.VectorSubcoreMesh(core_axis_name="c", subcore_axis_name="s")

    @functools.partial(
        pl.kernel,
        out_type=jax.ShapeDtypeStruct((B, D), jnp.float32),
        mesh=mesh,
        compiler_params=pltpu.CompilerParams(use_tc_tiling_on_sc=False),
        scratch_types=[
            pltpu.VMEM((g_per_chunk, 128), jnp.int32),
            pltpu.VMEM((rows_per_chunk, D), jnp.float32),
            pltpu.VMEM((ch, D), jnp.float32),
            pltpu.SemaphoreType.DMA,
        ],
    )
    def encode(idx_hbm, table_hbm, out_hbm, idx_v, rows_v, out_v, sem):
        wid = lax.axis_index("s") * nc + lax.axis_index("c")
        for c in range(chunks):
            base_pt = pl.multiple_of(wid * pw + c * ch, 8)
            gbase = pl.multiple_of((wid * pw * L + c * rows_per_chunk) // 128, 8)
            pltpu.sync_copy(idx_hbm.at[pl.ds(gbase, g_per_chunk)], idx_v)
            copies = [
                pltpu.async_copy(
                    table_hbm.at[idx_v.at[g]],
                    rows_v.at[pl.ds(g * 128, 128)],
                    sem,
                )
                for g in range(g_per_chunk)
            ]
            for cp in copies:
                cp.wait()

            def point_body(p, carry):
                rbase = p * L
                acc = jnp.zeros((LANES,), jnp.float32)
                for j in range(L):
                    v = rows_v[rbase + j]
                    acc = acc + v * _rsqrt_vec(
                        _allreduce(v * v, jnp.add), iters=1)
                s2 = _allreduce(acc * acc, jnp.add)
                out_v[p] = acc * _rsqrt_vec(s2)
                return carry

            lax.fori_loop(0, ch, point_body, 0)
            pltpu.sync_copy(out_v, out_hbm.at[pl.ds(base_pt, ch)])

    return encode


def kernel(indices, table):
    info = plsc.get_sparse_core_info()
    trans = _make_transpose(info.num_cores, info.num_subcores)
    enc = _make_encoder(info.num_cores, info.num_subcores)
    taux8 = table[VMAIN:].reshape(8, 128)
    trm = trans(table.T, taux8)
    trm2 = trm.reshape(VOCAB, D)
    idx_r = indices.astype(jnp.int32).reshape(B * L // 128, 128)
    return enc(idx_r, trm2)
